# baseline (speedup 1.0000x reference)
import functools

import jax
import jax.numpy as jnp
from jax import lax
from jax.experimental import pallas as pl
from jax.experimental.pallas import tpu as pltpu
from jax.experimental.pallas import tpu_sc as plsc

N = 10000
D = 128
DH = D // 2
E = 320000
NC = 2
NS = 16
NW = NC * NS
ROWS = E // 128
RPW = ROWS // NW
EXTRA = ROWS - RPW * NW
RPT = N // NS
ZC = 125
CC = 2000


def _seg_sum_builder(with_cnt):
  mesh = plsc.VectorSubcoreMesh(core_axis_name="c", subcore_axis_name="s")
  out_type = [jax.ShapeDtypeStruct((NC, 2, N, DH), jnp.float32)]
  scratch = [
      pltpu.VMEM((RPW, 128), jnp.int32),
      pltpu.VMEM((RPW, 128), jnp.int32),
      pltpu.VMEM((1, 128), jnp.int32),
      pltpu.VMEM((1, 128), jnp.int32),
      pltpu.VMEM((128, DH), jnp.float32),
      pltpu.VMEM((128, DH), jnp.float32),
      pltpu.VMEM_SHARED((N, DH), jnp.float32),
      pltpu.VMEM_SHARED((N, DH), jnp.float32),
      pltpu.SemaphoreType.DMA,
      pltpu.SemaphoreType.DMA,
      pltpu.SemaphoreType.DMA,
  ]
  if with_cnt:
    out_type.append(jax.ShapeDtypeStruct((NC, N), jnp.float32))
    scratch += [
        pltpu.VMEM((128,), jnp.float32),
        pltpu.VMEM((CC,), jnp.float32),
        pltpu.VMEM_SHARED((N,), jnp.float32),
    ]

  def body(xc_hbm, src_hbm, dst_hbm, agg_out, *rest):
    if with_cnt:
      (cnt_out, sidx, didx, esidx, edidx, rows, rows2, x_sh, agg_sh,
       sem, sem_s, sem_c, ones_v, cnt_v, cnt_sh) = rest
    else:
      (sidx, didx, esidx, edidx, rows, rows2, x_sh, agg_sh,
       sem, sem_s, sem_c) = rest
    c = lax.axis_index("c")
    s = lax.axis_index("s")
    w = c * NS + s
    zero16 = jnp.zeros((16,), jnp.float32)
    base = w * RPW

    pltpu.sync_copy(src_hbm.at[pl.ds(base, RPW)], sidx)
    pltpu.sync_copy(dst_hbm.at[pl.ds(base, RPW)], didx)
    @pl.when(w < EXTRA)
    def _():
      off = RPW * NW + w
      pltpu.sync_copy(src_hbm.at[pl.ds(off, 1)], esidx)
      pltpu.sync_copy(dst_hbm.at[pl.ds(off, 1)], edidx)

    @pl.loop(0, ZC)
    def _(r):
      for j in range(DH // 16):
        rows[r, pl.ds(j * 16, 16)] = zero16

    if with_cnt:
      one16 = jnp.ones((16,), jnp.float32)

      @pl.loop(0, 128 // 16)
      def _(i):
        ones_v[pl.ds(i * 16, 16)] = one16

      @pl.when(s == 0)
      def _():
        @pl.loop(0, CC // 16)
        def _(i):
          cnt_v[pl.ds(i * 16, 16)] = zero16
        for j in range(N // CC):
          pltpu.sync_copy(cnt_v, cnt_sh.at[pl.ds(j * CC, CC)])

    def g_start(j, buf):
      pltpu.async_copy(x_sh.at[sidx.at[j]], buf, sem)

    def g_wait(j, buf):
      pltpu.make_async_copy(x_sh.at[sidx.at[j]], buf, sem).wait()

    def s_start(j, buf):
      pltpu.async_copy(buf, agg_sh.at[didx.at[j]], sem_s, add=True)

    def s_wait(buf):
      pltpu.make_async_copy(buf, agg_sh.at[didx.at[0]], sem_s).wait()

    for h in range(2):
      for j in range(RPT // ZC):
        r0 = s * RPT + j * ZC
        pltpu.sync_copy(rows.at[pl.ds(0, ZC)], agg_sh.at[pl.ds(r0, ZC)])
        pltpu.sync_copy(xc_hbm.at[h, pl.ds(r0, ZC)], x_sh.at[pl.ds(r0, ZC)])
      plsc.subcore_barrier()

      g_start(0, rows)

      @pl.loop(0, RPW // 2)
      def _(t):
        j0 = 2 * t
        g_wait(j0, rows)
        @pl.when(t > 0)
        def _():
          s_wait(rows2)
        g_start(j0 + 1, rows2)
        s_start(j0, rows)
        if with_cnt and h == 0:
          pltpu.async_copy(ones_v, cnt_sh.at[didx.at[j0]], sem_c, add=True)
        g_wait(j0 + 1, rows2)
        s_wait(rows)
        @pl.when(t < RPW // 2 - 1)
        def _():
          g_start(j0 + 2, rows)
        s_start(j0 + 1, rows2)
        if with_cnt and h == 0:
          pltpu.async_copy(ones_v, cnt_sh.at[didx.at[j0 + 1]], sem_c,
                           add=True)

      s_wait(rows2)

      @pl.when(w < EXTRA)
      def _():
        pltpu.async_copy(x_sh.at[esidx.at[0]], rows, sem).wait()
        pltpu.sync_copy(rows, agg_sh.at[edidx.at[0]], add=True)
        if with_cnt and h == 0:
          pltpu.sync_copy(ones_v, cnt_sh.at[edidx.at[0]], add=True)

      if with_cnt and h == 0:
        for _ in range(2):
          pltpu.make_async_copy(xc_hbm.at[0, pl.ds(0, RPW)],
                                rows.at[pl.ds(0, RPW)], sem_c).wait()

      plsc.subcore_barrier()

      for j in range(RPT // ZC):
        r0 = s * RPT + j * ZC
        pltpu.sync_copy(agg_sh.at[pl.ds(r0, ZC)], rows2.at[pl.ds(0, ZC)])
        pltpu.sync_copy(rows2.at[pl.ds(0, ZC)],
                        agg_out.at[c, h, pl.ds(r0, ZC)])
      if h == 0:
        @pl.loop(0, ZC)
        def _(r):
          for j in range(DH // 16):
            rows[r, pl.ds(j * 16, 16)] = zero16

    if with_cnt:
      @pl.when(s == 0)
      def _():
        for j in range(N // CC):
          pltpu.sync_copy(cnt_sh.at[pl.ds(j * CC, CC)], cnt_v)
          pltpu.sync_copy(cnt_v, cnt_out.at[c, pl.ds(j * CC, CC)])

  return pl.kernel(
      body, out_type=out_type, mesh=mesh, scratch_types=scratch,
      compiler_params=pltpu.CompilerParams(use_tc_tiling_on_sc=False))


_seg_sum_cnt = _seg_sum_builder(True)
_seg_sum = _seg_sum_builder(False)

_BLK = 1000


def _layer_body(aggp, cntp, x, wlT, wrT, b, o, *, relu, split_in, split_out):
  agg = jnp.concatenate(
      (aggp[0, 0] + aggp[1, 0], aggp[0, 1] + aggp[1, 1]), axis=1)
  cnt = jnp.maximum(cntp[0] + cntp[1], 1.0)
  mean = agg / cnt
  if split_in:
    xin = jnp.concatenate((x[0], x[1]), axis=1)
  else:
    xin = x[...]
  h = (jnp.dot(mean, wlT[...], preferred_element_type=jnp.float32)
       + jnp.dot(xin, wrT[...], preferred_element_type=jnp.float32)
       + b[...])
  if relu:
    h = jnp.maximum(h, 0.0)
  if split_out:
    o[0] = h[:, :DH]
    o[1] = h[:, DH:]
  else:
    o[...] = h


def _layer(aggp, cntp, xin, wlT, wrT, b, relu, split_in, split_out):
  in_specs = [
      pl.BlockSpec((NC, 2, _BLK, DH), lambda i: (0, 0, i, 0)),
      pl.BlockSpec((NC, _BLK, 1), lambda i: (0, i, 0)),
      pl.BlockSpec((2, _BLK, DH), lambda i: (0, i, 0)) if split_in
      else pl.BlockSpec((_BLK, D), lambda i: (i, 0)),
      pl.BlockSpec((D, D), lambda i: (0, 0)),
      pl.BlockSpec((D, D), lambda i: (0, 0)),
      pl.BlockSpec((1, D), lambda i: (0, 0)),
  ]
  if split_out:
    out_spec = pl.BlockSpec((2, _BLK, DH), lambda i: (0, i, 0))
    out_shape = jax.ShapeDtypeStruct((2, N, DH), jnp.float32)
  else:
    out_spec = pl.BlockSpec((_BLK, D), lambda i: (i, 0))
    out_shape = jax.ShapeDtypeStruct((N, D), jnp.float32)
  return pl.pallas_call(
      functools.partial(_layer_body, relu=relu, split_in=split_in,
                        split_out=split_out),
      grid=(N // _BLK,),
      in_specs=in_specs,
      out_specs=out_spec,
      out_shape=out_shape,
  )(aggp, cntp, xin, wlT, wrT, b)


def kernel(x, edge_index, W1l, W1r, b1, W2l, W2r, b2):
  src = edge_index[0].reshape(ROWS, 128)
  dst = edge_index[1].reshape(ROWS, 128)
  xc = x.reshape(N, 2, DH).swapaxes(0, 1)
  agg1, cntp = _seg_sum_cnt(xc, src, dst)
  cntp = cntp.reshape(NC, N, 1)
  hc = _layer(agg1, cntp, x, W1l.T, W1r.T, b1.reshape(1, D), True,
              split_in=False, split_out=True)
  (agg2,) = _seg_sum(hc, src, dst)
  return _layer(agg2, cntp, hc, W2l.T, W2r.T, b2.reshape(1, D), False,
                split_in=True, split_out=False)


# speedup vs baseline: 1.3559x; 1.3559x over previous
import functools

import jax
import jax.numpy as jnp
from jax import lax
from jax.experimental import pallas as pl
from jax.experimental.pallas import tpu as pltpu
from jax.experimental.pallas import tpu_sc as plsc

N = 10000
D = 128
E = 320000
NC = 2
NS = 16
NW = NC * NS
ROWS = E // 128
RPW = ROWS // NW
EXTRA = ROWS - RPW * NW
PB = 26
NPH = RPW // PB
CC = 2000
RPT = N // NS
ZC = 125


def _seg_sum_builder(with_cnt):
  mesh = plsc.VectorSubcoreMesh(core_axis_name="c", subcore_axis_name="s")
  out_type = [jax.ShapeDtypeStruct((NC, N, D), jnp.float32)]
  scratch = [
      pltpu.VMEM((PB, 128), jnp.int32),
      pltpu.VMEM((PB, 128), jnp.int32),
      pltpu.VMEM((128, D), jnp.float32),
      pltpu.VMEM((128, D), jnp.float32),
      pltpu.VMEM_SHARED((N, D), jnp.float32),
      pltpu.SemaphoreType.DMA,
      pltpu.SemaphoreType.DMA,
      pltpu.SemaphoreType.DMA,
  ]
  if with_cnt:
    out_type.append(jax.ShapeDtypeStruct((NC, N), jnp.float32))
    scratch += [
        pltpu.VMEM((128,), jnp.float32),
        pltpu.VMEM((CC,), jnp.float32),
        pltpu.VMEM_SHARED((N,), jnp.float32),
    ]

  def body(x_hbm, src_hbm, dst_hbm, agg_out, *rest):
    if with_cnt:
      (cnt_out, sidx, didx, rows, rows2, agg_sh, sem, sem_s, sem_c,
       ones_v, cnt_v, cnt_sh) = rest
    else:
      (sidx, didx, rows, rows2, agg_sh, sem, sem_s, sem_c) = rest
    c = lax.axis_index("c")
    s = lax.axis_index("s")
    w = c * NS + s
    zero16 = jnp.zeros((16,), jnp.float32)

    @pl.loop(0, ZC)
    def _(r):
      for j in range(D // 16):
        rows[r, pl.ds(j * 16, 16)] = zero16

    for j in range(RPT // ZC):
      pltpu.sync_copy(rows.at[pl.ds(0, ZC)],
                      agg_sh.at[pl.ds(s * RPT + j * ZC, ZC)])

    if with_cnt:
      one16 = jnp.ones((16,), jnp.float32)

      @pl.loop(0, 128 // 16)
      def _(i):
        ones_v[pl.ds(i * 16, 16)] = one16

      @pl.when(s == 0)
      def _():
        @pl.loop(0, CC // 16)
        def _(i):
          cnt_v[pl.ds(i * 16, 16)] = zero16
        for j in range(N // CC):
          pltpu.sync_copy(cnt_v, cnt_sh.at[pl.ds(j * CC, CC)])

    plsc.subcore_barrier()

    base = w * RPW

    def g_start(j, buf):
      pltpu.async_copy(x_hbm.at[sidx.at[j]], buf, sem)

    def g_wait(j, buf):
      pltpu.make_async_copy(x_hbm.at[sidx.at[j]], buf, sem).wait()

    def s_start(j, buf):
      pltpu.async_copy(buf, agg_sh.at[didx.at[j]], sem_s, add=True)

    def s_wait(buf):
      pltpu.make_async_copy(buf, agg_sh.at[didx.at[0]], sem_s).wait()

    for ph in range(NPH):
      off = base + ph * PB
      pltpu.sync_copy(src_hbm.at[pl.ds(off, PB)], sidx)
      pltpu.sync_copy(dst_hbm.at[pl.ds(off, PB)], didx)
      g_start(0, rows)

      @pl.loop(0, PB // 2)
      def _(t):
        j0 = 2 * t
        g_wait(j0, rows)
        @pl.when(t > 0)
        def _():
          s_wait(rows2)
        g_start(j0 + 1, rows2)
        s_start(j0, rows)
        if with_cnt:
          pltpu.async_copy(ones_v, cnt_sh.at[didx.at[j0]], sem_c, add=True)
        g_wait(j0 + 1, rows2)
        s_wait(rows)
        @pl.when(t < PB // 2 - 1)
        def _():
          g_start(j0 + 2, rows)
        s_start(j0 + 1, rows2)
        if with_cnt:
          pltpu.async_copy(ones_v, cnt_sh.at[didx.at[j0 + 1]], sem_c,
                           add=True)

      s_wait(rows2)
      if with_cnt:
        pltpu.make_async_copy(x_hbm.at[pl.ds(0, PB)],
                              rows.at[pl.ds(0, PB)], sem_c).wait()

    @pl.when(w < EXTRA)
    def _():
      off = RPW * NW + w
      pltpu.sync_copy(src_hbm.at[pl.ds(off, 1)], sidx.at[pl.ds(0, 1)])
      pltpu.sync_copy(dst_hbm.at[pl.ds(off, 1)], didx.at[pl.ds(0, 1)])
      pltpu.async_copy(x_hbm.at[sidx.at[0]], rows, sem).wait()
      pltpu.sync_copy(rows, agg_sh.at[didx.at[0]], add=True)
      if with_cnt:
        pltpu.sync_copy(ones_v, cnt_sh.at[didx.at[0]], add=True)

    plsc.subcore_barrier()

    for j in range(RPT // ZC):
      r0 = s * RPT + j * ZC
      pltpu.sync_copy(agg_sh.at[pl.ds(r0, ZC)], rows.at[pl.ds(0, ZC)])
      pltpu.sync_copy(rows.at[pl.ds(0, ZC)], agg_out.at[c, pl.ds(r0, ZC)])
    if with_cnt:
      @pl.when(s == 0)
      def _():
        for j in range(N // CC):
          pltpu.sync_copy(cnt_sh.at[pl.ds(j * CC, CC)], cnt_v)
          pltpu.sync_copy(cnt_v, cnt_out.at[c, pl.ds(j * CC, CC)])

  return pl.kernel(
      body, out_type=out_type, mesh=mesh, scratch_types=scratch,
      compiler_params=pltpu.CompilerParams(use_tc_tiling_on_sc=False))


_seg_sum_cnt = _seg_sum_builder(True)
_seg_sum = _seg_sum_builder(False)

_BLK = 1000


def _layer_body(aggp, cntp, x, wlT, wrT, b, o, *, relu):
  agg = aggp[0] + aggp[1]
  cnt = jnp.maximum(cntp[0] + cntp[1], 1.0)
  mean = agg / cnt
  h = (jnp.dot(mean, wlT[...], preferred_element_type=jnp.float32)
       + jnp.dot(x[...], wrT[...], preferred_element_type=jnp.float32)
       + b[...])
  if relu:
    h = jnp.maximum(h, 0.0)
  o[...] = h


def _layer(aggp, cntp, xin, wlT, wrT, b, relu):
  return pl.pallas_call(
      functools.partial(_layer_body, relu=relu),
      grid=(N // _BLK,),
      in_specs=[
          pl.BlockSpec((NC, _BLK, D), lambda i: (0, i, 0)),
          pl.BlockSpec((NC, _BLK, 1), lambda i: (0, i, 0)),
          pl.BlockSpec((_BLK, D), lambda i: (i, 0)),
          pl.BlockSpec((D, D), lambda i: (0, 0)),
          pl.BlockSpec((D, D), lambda i: (0, 0)),
          pl.BlockSpec((1, D), lambda i: (0, 0)),
      ],
      out_specs=pl.BlockSpec((_BLK, D), lambda i: (i, 0)),
      out_shape=jax.ShapeDtypeStruct((N, D), jnp.float32),
  )(aggp, cntp, xin, wlT, wrT, b)


def kernel(x, edge_index, W1l, W1r, b1, W2l, W2r, b2):
  src = edge_index[0].reshape(ROWS, 128)
  dst = edge_index[1].reshape(ROWS, 128)
  agg1, cntp = _seg_sum_cnt(x, src, dst)
  cntp = cntp.reshape(NC, N, 1)
  h = _layer(agg1, cntp, x, W1l.T, W1r.T, b1.reshape(1, D), True)
  (agg2,) = _seg_sum(h, src, dst)
  return _layer(agg2, cntp, h, W2l.T, W2r.T, b2.reshape(1, D), False)


# speedup vs baseline: 1.3648x; 1.0065x over previous
import functools

import jax
import jax.numpy as jnp
from jax import lax
from jax.experimental import pallas as pl
from jax.experimental.pallas import tpu as pltpu
from jax.experimental.pallas import tpu_sc as plsc

N = 10000
D = 128
E = 320000
NC = 2
NS = 16
NW = NC * NS
ROWS = E // 128
RPW = ROWS // NW
EXTRA = ROWS - RPW * NW
PB = 26
NPH = RPW // PB
CC = 2000
RPT = N // NS
ZC = 125


def _seg_sum_builder(with_cnt):
  mesh = plsc.VectorSubcoreMesh(core_axis_name="c", subcore_axis_name="s")
  out_type = [jax.ShapeDtypeStruct((NC, N, D), jnp.float32)]
  scratch = [
      pltpu.VMEM((PB, 128), jnp.int32),
      pltpu.VMEM((PB, 128), jnp.int32),
      pltpu.VMEM((128, D), jnp.float32),
      pltpu.VMEM((128, D), jnp.float32),
      pltpu.VMEM_SHARED((N, D), jnp.float32),
      pltpu.SemaphoreType.DMA,
      pltpu.SemaphoreType.DMA,
      pltpu.SemaphoreType.DMA,
  ]
  if with_cnt:
    out_type.append(jax.ShapeDtypeStruct((NC, N), jnp.float32))
    scratch += [
        pltpu.VMEM((128,), jnp.float32),
        pltpu.VMEM((CC,), jnp.float32),
        pltpu.VMEM_SHARED((N,), jnp.float32),
    ]

  def body(x_hbm, src_hbm, dst_hbm, agg_out, *rest):
    if with_cnt:
      (cnt_out, sidx, didx, rows, rows2, agg_sh, sem, sem_s, sem_c,
       ones_v, cnt_v, cnt_sh) = rest
    else:
      (sidx, didx, rows, rows2, agg_sh, sem, sem_s, sem_c) = rest
    c = lax.axis_index("c")
    s = lax.axis_index("s")
    w = c * NS + s
    zero16 = jnp.zeros((16,), jnp.float32)

    @pl.loop(0, ZC)
    def _(r):
      for j in range(D // 16):
        rows[r, pl.ds(j * 16, 16)] = zero16

    for j in range(RPT // ZC):
      pltpu.sync_copy(rows.at[pl.ds(0, ZC)],
                      agg_sh.at[pl.ds(s * RPT + j * ZC, ZC)])

    if with_cnt:
      one16 = jnp.ones((16,), jnp.float32)

      @pl.loop(0, 128 // 16)
      def _(i):
        ones_v[pl.ds(i * 16, 16)] = one16

      @pl.when(s == 0)
      def _():
        @pl.loop(0, CC // 16)
        def _(i):
          cnt_v[pl.ds(i * 16, 16)] = zero16
        for j in range(N // CC):
          pltpu.sync_copy(cnt_v, cnt_sh.at[pl.ds(j * CC, CC)])

    plsc.subcore_barrier()

    base = w * RPW

    def g_start(j, buf):
      pltpu.async_copy(x_hbm.at[sidx.at[j]], buf, sem)

    def g_wait(j, buf):
      pltpu.make_async_copy(x_hbm.at[sidx.at[j]], buf, sem).wait()

    def s_start(j, buf):
      pltpu.async_copy(buf, agg_sh.at[didx.at[j]], sem_s, add=True)

    def s_wait(buf):
      pltpu.make_async_copy(buf, agg_sh.at[didx.at[0]], sem_s).wait()

    for ph in range(NPH):
      off = base + ph * PB
      pltpu.sync_copy(src_hbm.at[pl.ds(off, PB)], sidx)
      pltpu.sync_copy(dst_hbm.at[pl.ds(off, PB)], didx)
      g_start(0, rows)

      @pl.loop(0, PB // 2)
      def _(t):
        j0 = 2 * t
        g_wait(j0, rows)
        @pl.when(t > 0)
        def _():
          s_wait(rows2)
        g_start(j0 + 1, rows2)
        s_start(j0, rows)
        if with_cnt:
          pltpu.async_copy(ones_v, cnt_sh.at[didx.at[j0]], sem_c, add=True)
        g_wait(j0 + 1, rows2)
        s_wait(rows)
        @pl.when(t < PB // 2 - 1)
        def _():
          g_start(j0 + 2, rows)
        s_start(j0 + 1, rows2)
        if with_cnt:
          pltpu.async_copy(ones_v, cnt_sh.at[didx.at[j0 + 1]], sem_c,
                           add=True)

      s_wait(rows2)
      if with_cnt:
        pltpu.make_async_copy(x_hbm.at[pl.ds(0, PB)],
                              rows.at[pl.ds(0, PB)], sem_c).wait()

    @pl.when(w < EXTRA)
    def _():
      off = RPW * NW + w
      pltpu.sync_copy(src_hbm.at[pl.ds(off, 1)], sidx.at[pl.ds(0, 1)])
      pltpu.sync_copy(dst_hbm.at[pl.ds(off, 1)], didx.at[pl.ds(0, 1)])
      pltpu.async_copy(x_hbm.at[sidx.at[0]], rows, sem).wait()
      pltpu.sync_copy(rows, agg_sh.at[didx.at[0]], add=True)
      if with_cnt:
        pltpu.sync_copy(ones_v, cnt_sh.at[didx.at[0]], add=True)

    plsc.subcore_barrier()

    for j in range(RPT // ZC):
      r0 = s * RPT + j * ZC
      pltpu.sync_copy(agg_sh.at[pl.ds(r0, ZC)], rows.at[pl.ds(0, ZC)])
      pltpu.sync_copy(rows.at[pl.ds(0, ZC)], agg_out.at[c, pl.ds(r0, ZC)])
    if with_cnt:
      @pl.when(s == 0)
      def _():
        for j in range(N // CC):
          pltpu.sync_copy(cnt_sh.at[pl.ds(j * CC, CC)], cnt_v)
          pltpu.sync_copy(cnt_v, cnt_out.at[c, pl.ds(j * CC, CC)])

  return pl.kernel(
      body, out_type=out_type, mesh=mesh, scratch_types=scratch,
      compiler_params=pltpu.CompilerParams(use_tc_tiling_on_sc=False))


_seg_sum_cnt = _seg_sum_builder(True)
_seg_sum = _seg_sum_builder(False)

_BLK = 1000


def _matT(a, w):
  return jax.lax.dot_general(a, w, (((1,), (1,)), ((), ())),
                             preferred_element_type=jnp.float32)


def _root_body(x, wr, b, o):
  o[...] = _matT(x[...], wr[...]) + b[...]


def _root(xin, wr, b):
  return pl.pallas_call(
      _root_body,
      grid=(N // _BLK,),
      in_specs=[
          pl.BlockSpec((_BLK, D), lambda i: (i, 0)),
          pl.BlockSpec((D, D), lambda i: (0, 0)),
          pl.BlockSpec((1, D), lambda i: (0, 0)),
      ],
      out_specs=pl.BlockSpec((_BLK, D), lambda i: (i, 0)),
      out_shape=jax.ShapeDtypeStruct((N, D), jnp.float32),
  )(xin, wr, b)


def _layer_body(aggp, cntp, xr, wl, o, *, relu):
  agg = aggp[0] + aggp[1]
  cnt = jnp.maximum(cntp[0] + cntp[1], 1.0)
  mean = agg / cnt
  h = _matT(mean, wl[...]) + xr[...]
  if relu:
    h = jnp.maximum(h, 0.0)
  o[...] = h


def _layer(aggp, cntp, xr, wl, relu):
  return pl.pallas_call(
      functools.partial(_layer_body, relu=relu),
      grid=(N // _BLK,),
      in_specs=[
          pl.BlockSpec((NC, _BLK, D), lambda i: (0, i, 0)),
          pl.BlockSpec((NC, _BLK, 1), lambda i: (0, i, 0)),
          pl.BlockSpec((_BLK, D), lambda i: (i, 0)),
          pl.BlockSpec((D, D), lambda i: (0, 0)),
      ],
      out_specs=pl.BlockSpec((_BLK, D), lambda i: (i, 0)),
      out_shape=jax.ShapeDtypeStruct((N, D), jnp.float32),
  )(aggp, cntp, xr, wl)


def kernel(x, edge_index, W1l, W1r, b1, W2l, W2r, b2):
  src = edge_index[0].reshape(ROWS, 128)
  dst = edge_index[1].reshape(ROWS, 128)
  agg1, cntp = _seg_sum_cnt(x, src, dst)
  xr1 = _root(x, W1r, b1.reshape(1, D))
  cntp = cntp.reshape(NC, N, 1)
  h = _layer(agg1, cntp, xr1, W1l, True)
  (agg2,) = _seg_sum(h, src, dst)
  hr2 = _root(h, W2r, b2.reshape(1, D))
  return _layer(agg2, cntp, hr2, W2l, False)


# speedup vs baseline: 1.3998x; 1.0256x over previous
import functools

import jax
import jax.numpy as jnp
from jax import lax
from jax.experimental import pallas as pl
from jax.experimental.pallas import tpu as pltpu
from jax.experimental.pallas import tpu_sc as plsc

N = 10000
D = 128
E = 320000
NC = 2
NS = 16
NW = NC * NS
ROWS = E // 128
RPW = ROWS // NW
EXTRA = ROWS - RPW * NW
PB = 26
NPH = RPW // PB
CC = 2000
RPT = N // NS
ZC = 125


def _seg_sum_builder(with_cnt):
  mesh = plsc.VectorSubcoreMesh(core_axis_name="c", subcore_axis_name="s")
  out_type = [jax.ShapeDtypeStruct((NC, N, D), jnp.float32)]
  scratch = [
      pltpu.VMEM((PB, 128), jnp.int32),
      pltpu.VMEM((PB, 128), jnp.int32),
      pltpu.VMEM((PB, 128), jnp.int32),
      pltpu.VMEM((PB, 128), jnp.int32),
      pltpu.VMEM((128, D), jnp.float32),
      pltpu.VMEM((128, D), jnp.float32),
      pltpu.VMEM_SHARED((N, D), jnp.float32),
      pltpu.SemaphoreType.DMA,
      pltpu.SemaphoreType.DMA,
      pltpu.SemaphoreType.DMA,
      pltpu.SemaphoreType.DMA,
  ]
  if with_cnt:
    out_type.append(jax.ShapeDtypeStruct((NC, N), jnp.float32))
    scratch += [
        pltpu.VMEM((128,), jnp.float32),
        pltpu.VMEM((CC,), jnp.float32),
        pltpu.VMEM_SHARED((N,), jnp.float32),
    ]

  def body(x_hbm, src_hbm, dst_hbm, agg_out, *rest):
    if with_cnt:
      (cnt_out, sidx, didx, sidx2, didx2, rows, rows2, agg_sh,
       sem, sem_s, sem_c, sem_p, ones_v, cnt_v, cnt_sh) = rest
    else:
      (sidx, didx, sidx2, didx2, rows, rows2, agg_sh,
       sem, sem_s, sem_c, sem_p) = rest
    c = lax.axis_index("c")
    s = lax.axis_index("s")
    w = c * NS + s
    zero16 = jnp.zeros((16,), jnp.float32)
    base = w * RPW
    slabs = [(sidx, didx), (sidx2, didx2)]

    def slab_start(ph):
      off = base + ph * PB
      si, di = slabs[ph % 2]
      pltpu.async_copy(src_hbm.at[pl.ds(off, PB)], si, sem_p)
      pltpu.async_copy(dst_hbm.at[pl.ds(off, PB)], di, sem_p)

    def slab_wait(ph):
      off = base + ph * PB
      si, di = slabs[ph % 2]
      pltpu.make_async_copy(src_hbm.at[pl.ds(off, PB)], si, sem_p).wait()
      pltpu.make_async_copy(dst_hbm.at[pl.ds(off, PB)], di, sem_p).wait()

    slab_start(0)
    slab_start(1)

    @pl.loop(0, ZC)
    def _(r):
      for j in range(D // 16):
        rows[r, pl.ds(j * 16, 16)] = zero16

    for j in range(RPT // ZC):
      pltpu.sync_copy(rows.at[pl.ds(0, ZC)],
                      agg_sh.at[pl.ds(s * RPT + j * ZC, ZC)])

    if with_cnt:
      one16 = jnp.ones((16,), jnp.float32)

      @pl.loop(0, 128 // 16)
      def _(i):
        ones_v[pl.ds(i * 16, 16)] = one16

      @pl.when(s == 0)
      def _():
        @pl.loop(0, CC // 16)
        def _(i):
          cnt_v[pl.ds(i * 16, 16)] = zero16
        for j in range(N // CC):
          pltpu.sync_copy(cnt_v, cnt_sh.at[pl.ds(j * CC, CC)])

    plsc.subcore_barrier()

    for ph in range(NPH):
      si, di = slabs[ph % 2]

      def g_start(j, buf, si=si):
        pltpu.async_copy(x_hbm.at[si.at[j]], buf, sem)

      def g_wait(j, buf, si=si):
        pltpu.make_async_copy(x_hbm.at[si.at[j]], buf, sem).wait()

      def s_start(j, buf, di=di):
        pltpu.async_copy(buf, agg_sh.at[di.at[j]], sem_s, add=True)

      def s_wait(buf, di=di):
        pltpu.make_async_copy(buf, agg_sh.at[di.at[0]], sem_s).wait()

      slab_wait(ph)
      g_start(0, rows)

      @pl.loop(0, PB // 2)
      def _(t):
        j0 = 2 * t
        g_wait(j0, rows)
        @pl.when(t > 0)
        def _():
          s_wait(rows2)
        g_start(j0 + 1, rows2)
        s_start(j0, rows)
        if with_cnt:
          pltpu.async_copy(ones_v, cnt_sh.at[di.at[j0]], sem_c, add=True)
        g_wait(j0 + 1, rows2)
        s_wait(rows)
        @pl.when(t < PB // 2 - 1)
        def _():
          g_start(j0 + 2, rows)
        s_start(j0 + 1, rows2)
        if with_cnt:
          pltpu.async_copy(ones_v, cnt_sh.at[di.at[j0 + 1]], sem_c,
                           add=True)

      s_wait(rows2)
      if with_cnt:
        pltpu.make_async_copy(x_hbm.at[pl.ds(0, PB)],
                              rows.at[pl.ds(0, PB)], sem_c).wait()
      if ph + 2 < NPH:
        slab_start(ph + 2)

    @pl.when(w < EXTRA)
    def _():
      off = RPW * NW + w
      pltpu.sync_copy(src_hbm.at[pl.ds(off, 1)], sidx.at[pl.ds(0, 1)])
      pltpu.sync_copy(dst_hbm.at[pl.ds(off, 1)], didx.at[pl.ds(0, 1)])
      pltpu.async_copy(x_hbm.at[sidx.at[0]], rows, sem).wait()
      pltpu.sync_copy(rows, agg_sh.at[didx.at[0]], add=True)
      if with_cnt:
        pltpu.sync_copy(ones_v, cnt_sh.at[didx.at[0]], add=True)

    plsc.subcore_barrier()

    for j in range(RPT // ZC):
      r0 = s * RPT + j * ZC
      pltpu.sync_copy(agg_sh.at[pl.ds(r0, ZC)], rows.at[pl.ds(0, ZC)])
      pltpu.sync_copy(rows.at[pl.ds(0, ZC)], agg_out.at[c, pl.ds(r0, ZC)])
    if with_cnt:
      @pl.when(s == 0)
      def _():
        for j in range(N // CC):
          pltpu.sync_copy(cnt_sh.at[pl.ds(j * CC, CC)], cnt_v)
          pltpu.sync_copy(cnt_v, cnt_out.at[c, pl.ds(j * CC, CC)])

  return pl.kernel(
      body, out_type=out_type, mesh=mesh, scratch_types=scratch,
      compiler_params=pltpu.CompilerParams(use_tc_tiling_on_sc=False))


_seg_sum_cnt = _seg_sum_builder(True)
_seg_sum = _seg_sum_builder(False)

_BLK = 1000


def _matT(a, w):
  return jax.lax.dot_general(a, w, (((1,), (1,)), ((), ())),
                             preferred_element_type=jnp.float32)


def _root_body(x, wr, b, o):
  o[...] = _matT(x[...], wr[...]) + b[...]


def _root(xin, wr, b):
  return pl.pallas_call(
      _root_body,
      grid=(N // _BLK,),
      in_specs=[
          pl.BlockSpec((_BLK, D), lambda i: (i, 0)),
          pl.BlockSpec((D, D), lambda i: (0, 0)),
          pl.BlockSpec((1, D), lambda i: (0, 0)),
      ],
      out_specs=pl.BlockSpec((_BLK, D), lambda i: (i, 0)),
      out_shape=jax.ShapeDtypeStruct((N, D), jnp.float32),
  )(xin, wr, b)


def _layer_body(aggp, cntp, xr, wl, o, *, relu):
  agg = aggp[0] + aggp[1]
  cnt = jnp.maximum(cntp[0] + cntp[1], 1.0)
  mean = agg / cnt
  h = _matT(mean, wl[...]) + xr[...]
  if relu:
    h = jnp.maximum(h, 0.0)
  o[...] = h


def _layer(aggp, cntp, xr, wl, relu):
  return pl.pallas_call(
      functools.partial(_layer_body, relu=relu),
      grid=(N // _BLK,),
      in_specs=[
          pl.BlockSpec((NC, _BLK, D), lambda i: (0, i, 0)),
          pl.BlockSpec((NC, _BLK, 1), lambda i: (0, i, 0)),
          pl.BlockSpec((_BLK, D), lambda i: (i, 0)),
          pl.BlockSpec((D, D), lambda i: (0, 0)),
      ],
      out_specs=pl.BlockSpec((_BLK, D), lambda i: (i, 0)),
      out_shape=jax.ShapeDtypeStruct((N, D), jnp.float32),
  )(aggp, cntp, xr, wl)


def kernel(x, edge_index, W1l, W1r, b1, W2l, W2r, b2):
  src = edge_index[0].reshape(ROWS, 128)
  dst = edge_index[1].reshape(ROWS, 128)
  agg1, cntp = _seg_sum_cnt(x, src, dst)
  xr1 = _root(x, W1r, b1.reshape(1, D))
  cntp = cntp.reshape(NC, N, 1)
  h = _layer(agg1, cntp, xr1, W1l, True)
  (agg2,) = _seg_sum(h, src, dst)
  hr2 = _root(h, W2r, b2.reshape(1, D))
  return _layer(agg2, cntp, hr2, W2l, False)
